# Initial kernel scaffold; baseline (speedup 1.0000x reference)
import functools

import jax
import jax.numpy as jnp
from jax import lax
from jax.experimental import pallas as pl
from jax.experimental.pallas import tpu as pltpu
from jax.experimental.pallas import tpu_sc as plsc

N = 10000
D = 128
E = 320000

NC = 2
NS = 16
NW = NC * NS
CH = 128
NCHUNK = E // CH
NPAD = 10240

_VMESH = plsc.VectorSubcoreMesh(core_axis_name="core", subcore_axis_name="subcore")


def _sc_degrees(edge_index, zeros_col, ones_col):
    @pl.kernel(
        out_type=jax.ShapeDtypeStruct((NC, 2, NPAD, 1), jnp.float32),
        mesh=_VMESH,
        scratch_types=[
            pltpu.VMEM_SHARED((NPAD, 1), jnp.float32),
            pltpu.VMEM_SHARED((NPAD, 1), jnp.float32),
            pltpu.VMEM((CH,), jnp.int32),
            pltpu.VMEM((CH,), jnp.int32),
            pltpu.VMEM((CH, 1), jnp.float32),
        ],
    )
    def deg_kernel(edge_hbm, zero_hbm, one_hbm, out_hbm, hs_sh, hd_sh,
                   sidx, didx, ones_v):
        c = lax.axis_index("core")
        s = lax.axis_index("subcore")
        wid = c * NS + s
        rows = NPAD // NS
        pltpu.sync_copy(zero_hbm, hs_sh.at[pl.ds(s * rows, rows)])
        pltpu.sync_copy(zero_hbm, hd_sh.at[pl.ds(s * rows, rows)])
        pltpu.sync_copy(one_hbm, ones_v)
        plsc.subcore_barrier()

        n_k = jnp.where(wid < NCHUNK - (NCHUNK // NW) * NW, NCHUNK // NW + 1,
                        NCHUNK // NW)

        def body(k, carry):
            base = (wid + k * NW) * CH
            pltpu.sync_copy(edge_hbm.at[0, pl.ds(base, CH)], sidx)
            pltpu.sync_copy(edge_hbm.at[1, pl.ds(base, CH)], didx)
            pltpu.sync_copy(ones_v, hs_sh.at[sidx], add=True)
            pltpu.sync_copy(ones_v, hd_sh.at[didx], add=True)
            return carry

        lax.fori_loop(0, n_k, body, 0)
        plsc.subcore_barrier()
        pltpu.sync_copy(hs_sh.at[pl.ds(s * rows, rows)],
                        out_hbm.at[c, 0, pl.ds(s * rows, rows)])
        pltpu.sync_copy(hd_sh.at[pl.ds(s * rows, rows)],
                        out_hbm.at[c, 1, pl.ds(s * rows, rows)])

    return deg_kernel(edge_index, zeros_col, ones_col)


def _sc_aggregate(table, edge_index, zeros_rows):
    @pl.kernel(
        out_type=jax.ShapeDtypeStruct((NC, N, D), jnp.float32),
        mesh=_VMESH,
        scratch_types=[
            pltpu.VMEM_SHARED((N, D), jnp.float32),
            pltpu.VMEM((CH,), jnp.int32),
            pltpu.VMEM((CH,), jnp.int32),
            pltpu.VMEM((CH, D), jnp.float32),
        ],
    )
    def agg_kernel(table_hbm, edge_hbm, zero_hbm, out_hbm, acc_sh,
                   sidx, didx, rows_v):
        c = lax.axis_index("core")
        s = lax.axis_index("subcore")
        wid = c * NS + s
        rows = N // NS
        pltpu.sync_copy(zero_hbm, acc_sh.at[pl.ds(s * rows, rows)])
        plsc.subcore_barrier()

        n_k = jnp.where(wid < NCHUNK - (NCHUNK // NW) * NW, NCHUNK // NW + 1,
                        NCHUNK // NW)

        def body(k, carry):
            base = (wid + k * NW) * CH
            pltpu.sync_copy(edge_hbm.at[0, pl.ds(base, CH)], sidx)
            pltpu.sync_copy(edge_hbm.at[1, pl.ds(base, CH)], didx)
            pltpu.sync_copy(table_hbm.at[sidx], rows_v)
            pltpu.sync_copy(rows_v, acc_sh.at[didx], add=True)
            return carry

        lax.fori_loop(0, n_k, body, 0)
        plsc.subcore_barrier()
        pltpu.sync_copy(acc_sh.at[pl.ds(s * rows, rows)],
                        out_hbm.at[c, pl.ds(s * rows, rows)])

    return agg_kernel(table, edge_index, zeros_rows)


_BR = 1000


def _tc_first_body(deg_ref, x_ref, w_ref, h_ref, ns_ref, nd_ref):
    degp = deg_ref[...]
    dsrc = degp[0, 0] + degp[1, 0]
    ddst = degp[0, 1] + degp[1, 1]
    ns = lax.rsqrt(jnp.maximum(dsrc, 1.0))
    nd = lax.rsqrt(jnp.maximum(ddst, 1.0))
    h = jnp.dot(x_ref[...], w_ref[...], preferred_element_type=jnp.float32)
    h_ref[...] = h * ns
    ns_ref[...] = ns
    nd_ref[...] = nd


def _tc_first(deg_p, x, W1):
    grid = (N // _BR,)
    return pl.pallas_call(
        _tc_first_body,
        grid=grid,
        in_specs=[
            pl.BlockSpec((NC, 2, _BR, 1), lambda i: (0, 0, i, 0)),
            pl.BlockSpec((_BR, D), lambda i: (i, 0)),
            pl.BlockSpec((D, D), lambda i: (0, 0)),
        ],
        out_specs=[
            pl.BlockSpec((_BR, D), lambda i: (i, 0)),
            pl.BlockSpec((_BR, 1), lambda i: (i, 0)),
            pl.BlockSpec((_BR, 1), lambda i: (i, 0)),
        ],
        out_shape=[
            jax.ShapeDtypeStruct((N, D), jnp.float32),
            jax.ShapeDtypeStruct((N, 1), jnp.float32),
            jax.ShapeDtypeStruct((N, 1), jnp.float32),
        ],
    )(deg_p, x, W1)


def _tc_mid_body(p_ref, nd_ref, ns_ref, w_ref, b_ref, h_ref):
    agg = p_ref[0] + p_ref[1]
    x1 = jnp.maximum(agg * nd_ref[...] + b_ref[...], 0.0)
    h = jnp.dot(x1, w_ref[...], preferred_element_type=jnp.float32)
    h_ref[...] = h * ns_ref[...]


def _tc_mid(p1, nd, ns, W2, b1_row):
    grid = (N // _BR,)
    return pl.pallas_call(
        _tc_mid_body,
        grid=grid,
        in_specs=[
            pl.BlockSpec((NC, _BR, D), lambda i: (0, i, 0)),
            pl.BlockSpec((_BR, 1), lambda i: (i, 0)),
            pl.BlockSpec((_BR, 1), lambda i: (i, 0)),
            pl.BlockSpec((D, D), lambda i: (0, 0)),
            pl.BlockSpec((1, D), lambda i: (0, 0)),
        ],
        out_specs=pl.BlockSpec((_BR, D), lambda i: (i, 0)),
        out_shape=jax.ShapeDtypeStruct((N, D), jnp.float32),
    )(p1, nd, ns, W2, b1_row)


def _tc_final_body(p_ref, nd_ref, b_ref, o_ref):
    o_ref[...] = (p_ref[0] + p_ref[1]) * nd_ref[...] + b_ref[...]


def _tc_final(p2, nd, b2_row):
    grid = (N // _BR,)
    return pl.pallas_call(
        _tc_final_body,
        grid=grid,
        in_specs=[
            pl.BlockSpec((NC, _BR, D), lambda i: (0, i, 0)),
            pl.BlockSpec((_BR, 1), lambda i: (i, 0)),
            pl.BlockSpec((1, D), lambda i: (0, 0)),
        ],
        out_specs=pl.BlockSpec((_BR, D), lambda i: (i, 0)),
        out_shape=jax.ShapeDtypeStruct((N, D), jnp.float32),
    )(p2, nd, b2_row)


def kernel(edge_index, node_embeddings, W1, b1, W2, b2):
    zeros_col = jnp.zeros((NPAD // NS, 1), jnp.float32)
    ones_col = jnp.ones((CH, 1), jnp.float32)
    zeros_rows = jnp.zeros((N // NS, D), jnp.float32)

    deg_p = _sc_degrees(edge_index, zeros_col, ones_col)
    h1, ns, nd = _tc_first(deg_p, node_embeddings, W1)
    p1 = _sc_aggregate(h1, edge_index, zeros_rows)
    h2 = _tc_mid(p1, nd, ns, W2, jnp.reshape(b1, (1, D)))
    p2 = _sc_aggregate(h2, edge_index, zeros_rows)
    return _tc_final(p2, nd, jnp.reshape(b2, (1, D)))


# speedup vs baseline: 9.7454x; 9.7454x over previous
import dataclasses
import functools

import jax
import jax.numpy as jnp
from jax import lax
from jax.experimental import pallas as pl
from jax.experimental.pallas import tpu as pltpu
from jax.experimental.pallas import tpu_sc as plsc

N = 10000
D = 128
E = 320000

NC = 2
NS = 16
NW = NC * NS
CH = 128
NCHUNK = E // CH
NPAD = 10240
DEGW = 16

@functools.cache
def _vmesh():
    return plsc.VectorSubcoreMesh(core_axis_name="core", subcore_axis_name="subcore")


def _sc_params():
    cp = pltpu.CompilerParams()
    if "needs_layout_passes" in pltpu.CompilerParams.__dataclass_fields__:
        cp = dataclasses.replace(cp, needs_layout_passes=False)
    return cp


def _sc_degrees(edge_index, zeros_col):
    @pl.kernel(
        out_type=jax.ShapeDtypeStruct((2, NW, NPAD), jnp.float32),
        mesh=_vmesh(),
        compiler_params=_sc_params(),
        scratch_types=[
            pltpu.VMEM((NPAD,), jnp.float32),
            pltpu.VMEM((NPAD,), jnp.float32),
            pltpu.VMEM((CH,), jnp.int32),
            pltpu.VMEM((CH,), jnp.int32),
        ],
    )
    def deg_kernel(edge_hbm, zero_hbm, out_hbm, hs, hd, sidx, didx):
        c = lax.axis_index("core")
        s = lax.axis_index("subcore")
        wid = c * NS + s
        pltpu.sync_copy(zero_hbm, hs)
        pltpu.sync_copy(zero_hbm, hd)
        ones16 = jnp.ones((16,), jnp.float32)

        n_k = jnp.where(wid < NCHUNK - (NCHUNK // NW) * NW, NCHUNK // NW + 1,
                        NCHUNK // NW)

        def body(k, carry):
            base = (wid + k * NW) * CH
            pltpu.sync_copy(edge_hbm.at[0, pl.ds(base, CH)], sidx)
            pltpu.sync_copy(edge_hbm.at[1, pl.ds(base, CH)], didx)
            for j in range(CH // 16):
                si = sidx[pl.ds(j * 16, 16)]
                di = didx[pl.ds(j * 16, 16)]
                plsc.addupdate_scatter(hs, [si], ones16)
                plsc.addupdate_scatter(hd, [di], ones16)
            return carry

        lax.fori_loop(0, n_k, body, 0)
        pltpu.sync_copy(hs, out_hbm.at[0, wid])
        pltpu.sync_copy(hd, out_hbm.at[1, wid])

    return deg_kernel(edge_index, zeros_col)


def _sc_aggregate(table, edge_index, zeros_rows):
    @pl.kernel(
        out_type=jax.ShapeDtypeStruct((NC, NPAD, D), jnp.float32),
        mesh=_vmesh(),
        scratch_types=[
            pltpu.VMEM_SHARED((NPAD, D), jnp.float32),
            pltpu.VMEM((CH,), jnp.int32),
            pltpu.VMEM((CH,), jnp.int32),
            pltpu.VMEM((CH, D), jnp.float32),
        ],
    )
    def agg_kernel(table_hbm, edge_hbm, zero_hbm, out_hbm, acc_sh,
                   sidx, didx, rows_v):
        c = lax.axis_index("core")
        s = lax.axis_index("subcore")
        wid = c * NS + s
        rows = NPAD // NS
        pltpu.sync_copy(zero_hbm, acc_sh.at[pl.ds(s * rows, rows)])
        plsc.subcore_barrier()

        n_k = jnp.where(wid < NCHUNK - (NCHUNK // NW) * NW, NCHUNK // NW + 1,
                        NCHUNK // NW)

        def body(k, carry):
            base = (wid + k * NW) * CH
            pltpu.sync_copy(edge_hbm.at[0, pl.ds(base, CH)], sidx)
            pltpu.sync_copy(edge_hbm.at[1, pl.ds(base, CH)], didx)
            pltpu.sync_copy(table_hbm.at[sidx], rows_v)
            pltpu.sync_copy(rows_v, acc_sh.at[didx], add=True)
            return carry

        lax.fori_loop(0, n_k, body, 0)
        plsc.subcore_barrier()
        pltpu.sync_copy(acc_sh.at[pl.ds(s * rows, rows)],
                        out_hbm.at[c, pl.ds(s * rows, rows)])

    return agg_kernel(table, edge_index, zeros_rows)


_BR = 1000


def _tc_first_body(deg_ref, x_ref, w_ref, h_ref, ns_ref, nd_ref):
    degp = deg_ref[...]
    dsrc = jnp.sum(degp[0], axis=1, keepdims=True)
    ddst = jnp.sum(degp[1], axis=1, keepdims=True)
    ns = lax.rsqrt(jnp.maximum(dsrc, 1.0))
    nd = lax.rsqrt(jnp.maximum(ddst, 1.0))
    h = jnp.dot(x_ref[...], w_ref[...], preferred_element_type=jnp.float32)
    h_ref[...] = h * ns
    ns_ref[...] = ns
    nd_ref[...] = nd


def _tc_first(deg_p, x, W1):
    grid = (N // _BR,)
    return pl.pallas_call(
        _tc_first_body,
        grid=grid,
        in_specs=[
            pl.BlockSpec((2, _BR, NW), lambda i: (0, i, 0)),
            pl.BlockSpec((_BR, D), lambda i: (i, 0)),
            pl.BlockSpec((D, D), lambda i: (0, 0)),
        ],
        out_specs=[
            pl.BlockSpec((_BR, D), lambda i: (i, 0)),
            pl.BlockSpec((_BR, 1), lambda i: (i, 0)),
            pl.BlockSpec((_BR, 1), lambda i: (i, 0)),
        ],
        out_shape=[
            jax.ShapeDtypeStruct((N, D), jnp.float32),
            jax.ShapeDtypeStruct((N, 1), jnp.float32),
            jax.ShapeDtypeStruct((N, 1), jnp.float32),
        ],
    )(deg_p, x, W1)


def _tc_mid_body(p_ref, nd_ref, ns_ref, w_ref, b_ref, h_ref):
    agg = p_ref[0] + p_ref[1]
    x1 = jnp.maximum(agg * nd_ref[...] + b_ref[...], 0.0)
    h = jnp.dot(x1, w_ref[...], preferred_element_type=jnp.float32)
    h_ref[...] = h * ns_ref[...]


def _tc_mid(p1, nd, ns, W2, b1_row):
    grid = (N // _BR,)
    return pl.pallas_call(
        _tc_mid_body,
        grid=grid,
        in_specs=[
            pl.BlockSpec((NC, _BR, D), lambda i: (0, i, 0)),
            pl.BlockSpec((_BR, 1), lambda i: (i, 0)),
            pl.BlockSpec((_BR, 1), lambda i: (i, 0)),
            pl.BlockSpec((D, D), lambda i: (0, 0)),
            pl.BlockSpec((1, D), lambda i: (0, 0)),
        ],
        out_specs=pl.BlockSpec((_BR, D), lambda i: (i, 0)),
        out_shape=jax.ShapeDtypeStruct((N, D), jnp.float32),
    )(p1, nd, ns, W2, b1_row)


def _tc_final_body(p_ref, nd_ref, b_ref, o_ref):
    o_ref[...] = (p_ref[0] + p_ref[1]) * nd_ref[...] + b_ref[...]


def _tc_final(p2, nd, b2_row):
    grid = (N // _BR,)
    return pl.pallas_call(
        _tc_final_body,
        grid=grid,
        in_specs=[
            pl.BlockSpec((NC, _BR, D), lambda i: (0, i, 0)),
            pl.BlockSpec((_BR, 1), lambda i: (i, 0)),
            pl.BlockSpec((1, D), lambda i: (0, 0)),
        ],
        out_specs=pl.BlockSpec((_BR, D), lambda i: (i, 0)),
        out_shape=jax.ShapeDtypeStruct((N, D), jnp.float32),
    )(p2, nd, b2_row)


def kernel(edge_index, node_embeddings, W1, b1, W2, b2):
    zeros_col = jnp.zeros((NPAD,), jnp.float32)
    zeros_rows = jnp.zeros((NPAD // NS, D), jnp.float32)

    deg_p = jnp.transpose(_sc_degrees(edge_index, zeros_col), (0, 2, 1))
    h1, ns, nd = _tc_first(deg_p, node_embeddings, W1)
    p1 = _sc_aggregate(h1, edge_index, zeros_rows)
    h2 = _tc_mid(p1, nd, ns, W2, jnp.reshape(b1, (1, D)))
    p2 = _sc_aggregate(h2, edge_index, zeros_rows)
    return _tc_final(p2, nd, jnp.reshape(b2, (1, D)))


# speedup vs baseline: 12.8346x; 1.3170x over previous
import dataclasses
import functools

import jax
import jax.numpy as jnp
from jax import lax
from jax.experimental import pallas as pl
from jax.experimental.pallas import tpu as pltpu
from jax.experimental.pallas import tpu_sc as plsc

N = 10000
D = 128
E = 320000

NC = 2
NS = 16
NW = NC * NS
CH = 128
NCHUNK = E // CH
NPAD = 10240
DEGW = 16

@functools.cache
def _vmesh():
    return plsc.VectorSubcoreMesh(core_axis_name="core", subcore_axis_name="subcore")


def _sc_params():
    cp = pltpu.CompilerParams()
    if "needs_layout_passes" in pltpu.CompilerParams.__dataclass_fields__:
        cp = dataclasses.replace(cp, needs_layout_passes=False)
    return cp


def _sc_degrees(edge_index, zeros_col):
    @pl.kernel(
        out_type=jax.ShapeDtypeStruct((2, NW, NPAD), jnp.float32),
        mesh=_vmesh(),
        compiler_params=_sc_params(),
        scratch_types=[
            pltpu.VMEM((NPAD,), jnp.float32),
            pltpu.VMEM((NPAD,), jnp.float32),
            pltpu.VMEM((CH,), jnp.int32),
            pltpu.VMEM((CH,), jnp.int32),
        ],
    )
    def deg_kernel(edge_hbm, zero_hbm, out_hbm, hs, hd, sidx, didx):
        c = lax.axis_index("core")
        s = lax.axis_index("subcore")
        wid = c * NS + s
        pltpu.sync_copy(zero_hbm, hs)
        pltpu.sync_copy(zero_hbm, hd)
        ones16 = jnp.ones((16,), jnp.float32)

        n_k = jnp.where(wid < NCHUNK - (NCHUNK // NW) * NW, NCHUNK // NW + 1,
                        NCHUNK // NW)

        def body(k, carry):
            base = (wid + k * NW) * CH
            pltpu.sync_copy(edge_hbm.at[0, pl.ds(base, CH)], sidx)
            pltpu.sync_copy(edge_hbm.at[1, pl.ds(base, CH)], didx)
            for j in range(CH // 16):
                si = sidx[pl.ds(j * 16, 16)]
                di = didx[pl.ds(j * 16, 16)]
                plsc.addupdate_scatter(hs, [si], ones16)
                plsc.addupdate_scatter(hd, [di], ones16)
            return carry

        lax.fori_loop(0, n_k, body, 0)
        pltpu.sync_copy(hs, out_hbm.at[0, wid])
        pltpu.sync_copy(hd, out_hbm.at[1, wid])

    return deg_kernel(edge_index, zeros_col)


def _sc_aggregate(table, edge_index, zeros_rows):
    NFULL = (E // NW) // CH
    TAIL = E // NW - NFULL * CH

    @pl.kernel(
        out_type=jax.ShapeDtypeStruct((NC, NPAD, D), jnp.float32),
        mesh=_vmesh(),
        scratch_types=[
            pltpu.VMEM_SHARED((NPAD, D), jnp.float32),
            pltpu.VMEM((CH,), jnp.int32),
            pltpu.VMEM((CH,), jnp.int32),
            pltpu.VMEM((CH,), jnp.int32),
            pltpu.VMEM((CH,), jnp.int32),
            pltpu.VMEM((CH, D), jnp.float32),
            pltpu.VMEM((CH, D), jnp.float32),
            pltpu.VMEM((TAIL,), jnp.int32),
            pltpu.VMEM((TAIL,), jnp.int32),
            pltpu.VMEM((TAIL, D), jnp.float32),
            pltpu.SemaphoreType.DMA,
            pltpu.SemaphoreType.DMA,
            pltpu.SemaphoreType.DMA,
            pltpu.SemaphoreType.DMA,
        ],
    )
    def agg_kernel(table_hbm, edge_hbm, zero_hbm, out_hbm, acc_sh,
                   sidx0, didx0, sidx1, didx1, rows0, rows1,
                   sidxt, didxt, rowst, semi0, semi1, semg0, semg1):
        c = lax.axis_index("core")
        s = lax.axis_index("subcore")
        wid = c * NS + s
        rows = NPAD // NS
        pltpu.sync_copy(zero_hbm, acc_sh.at[pl.ds(s * rows, rows)])
        plsc.subcore_barrier()

        base_w = wid * (E // NW)

        def body(g, carry):
            b0 = base_w + (2 * g) * CH
            b1 = b0 + CH
            da0 = pltpu.async_copy(edge_hbm.at[pl.ds(b0, CH)], sidx0, semi0)
            db0 = pltpu.async_copy(edge_hbm.at[pl.ds(E + b0, CH)], didx0, semi0)
            da1 = pltpu.async_copy(edge_hbm.at[pl.ds(b1, CH)], sidx1, semi1)
            db1 = pltpu.async_copy(edge_hbm.at[pl.ds(E + b1, CH)], didx1, semi1)
            da0.wait()
            db0.wait()
            g0 = pltpu.async_copy(table_hbm.at[sidx0], rows0, semg0)
            da1.wait()
            db1.wait()
            g1 = pltpu.async_copy(table_hbm.at[sidx1], rows1, semg1)
            g0.wait()
            pltpu.sync_copy(rows0, acc_sh.at[didx0], add=True)
            g1.wait()
            pltpu.sync_copy(rows1, acc_sh.at[didx1], add=True)
            return carry

        lax.fori_loop(0, NFULL // 2, body, 0)

        bt = base_w + NFULL * CH
        pltpu.sync_copy(edge_hbm.at[pl.ds(bt, TAIL)], sidxt)
        pltpu.sync_copy(edge_hbm.at[pl.ds(E + bt, TAIL)], didxt)
        pltpu.sync_copy(table_hbm.at[sidxt], rowst)
        pltpu.sync_copy(rowst, acc_sh.at[didxt], add=True)

        plsc.subcore_barrier()
        pltpu.sync_copy(acc_sh.at[pl.ds(s * rows, rows)],
                        out_hbm.at[c, pl.ds(s * rows, rows)])

    return agg_kernel(table, jnp.reshape(edge_index, (2 * E,)), zeros_rows)


_BR = 1000


def _tc_first_body(deg_ref, x_ref, w_ref, h_ref, ns_ref, nd_ref):
    degp = deg_ref[...]
    dsrc = jnp.sum(degp[0], axis=1, keepdims=True)
    ddst = jnp.sum(degp[1], axis=1, keepdims=True)
    ns = lax.rsqrt(jnp.maximum(dsrc, 1.0))
    nd = lax.rsqrt(jnp.maximum(ddst, 1.0))
    h = jnp.dot(x_ref[...], w_ref[...], preferred_element_type=jnp.float32)
    h_ref[...] = h * ns
    ns_ref[...] = ns
    nd_ref[...] = nd


def _tc_first(deg_p, x, W1):
    grid = (N // _BR,)
    return pl.pallas_call(
        _tc_first_body,
        grid=grid,
        in_specs=[
            pl.BlockSpec((2, _BR, NW), lambda i: (0, i, 0)),
            pl.BlockSpec((_BR, D), lambda i: (i, 0)),
            pl.BlockSpec((D, D), lambda i: (0, 0)),
        ],
        out_specs=[
            pl.BlockSpec((_BR, D), lambda i: (i, 0)),
            pl.BlockSpec((_BR, 1), lambda i: (i, 0)),
            pl.BlockSpec((_BR, 1), lambda i: (i, 0)),
        ],
        out_shape=[
            jax.ShapeDtypeStruct((N, D), jnp.float32),
            jax.ShapeDtypeStruct((N, 1), jnp.float32),
            jax.ShapeDtypeStruct((N, 1), jnp.float32),
        ],
    )(deg_p, x, W1)


def _tc_mid_body(p_ref, nd_ref, ns_ref, w_ref, b_ref, h_ref):
    agg = p_ref[0] + p_ref[1]
    x1 = jnp.maximum(agg * nd_ref[...] + b_ref[...], 0.0)
    h = jnp.dot(x1, w_ref[...], preferred_element_type=jnp.float32)
    h_ref[...] = h * ns_ref[...]


def _tc_mid(p1, nd, ns, W2, b1_row):
    grid = (N // _BR,)
    return pl.pallas_call(
        _tc_mid_body,
        grid=grid,
        in_specs=[
            pl.BlockSpec((NC, _BR, D), lambda i: (0, i, 0)),
            pl.BlockSpec((_BR, 1), lambda i: (i, 0)),
            pl.BlockSpec((_BR, 1), lambda i: (i, 0)),
            pl.BlockSpec((D, D), lambda i: (0, 0)),
            pl.BlockSpec((1, D), lambda i: (0, 0)),
        ],
        out_specs=pl.BlockSpec((_BR, D), lambda i: (i, 0)),
        out_shape=jax.ShapeDtypeStruct((N, D), jnp.float32),
    )(p1, nd, ns, W2, b1_row)


def _tc_final_body(p_ref, nd_ref, b_ref, o_ref):
    o_ref[...] = (p_ref[0] + p_ref[1]) * nd_ref[...] + b_ref[...]


def _tc_final(p2, nd, b2_row):
    grid = (N // _BR,)
    return pl.pallas_call(
        _tc_final_body,
        grid=grid,
        in_specs=[
            pl.BlockSpec((NC, _BR, D), lambda i: (0, i, 0)),
            pl.BlockSpec((_BR, 1), lambda i: (i, 0)),
            pl.BlockSpec((1, D), lambda i: (0, 0)),
        ],
        out_specs=pl.BlockSpec((_BR, D), lambda i: (i, 0)),
        out_shape=jax.ShapeDtypeStruct((N, D), jnp.float32),
    )(p2, nd, b2_row)


def kernel(edge_index, node_embeddings, W1, b1, W2, b2):
    zeros_col = jnp.zeros((NPAD,), jnp.float32)
    zeros_rows = jnp.zeros((NPAD // NS, D), jnp.float32)

    deg_p = jnp.transpose(_sc_degrees(edge_index, zeros_col), (0, 2, 1))
    h1, ns, nd = _tc_first(deg_p, node_embeddings, W1)
    p1 = _sc_aggregate(h1, edge_index, zeros_rows)
    h2 = _tc_mid(p1, nd, ns, W2, jnp.reshape(b1, (1, D)))
    p2 = _sc_aggregate(h2, edge_index, zeros_rows)
    return _tc_final(p2, nd, jnp.reshape(b2, (1, D)))


# speedup vs baseline: 16.1258x; 1.2564x over previous
import dataclasses
import functools

import jax
import jax.numpy as jnp
from jax import lax
from jax.experimental import pallas as pl
from jax.experimental.pallas import tpu as pltpu
from jax.experimental.pallas import tpu_sc as plsc

N = 10000
D = 128
E = 320000

NC = 2
NS = 16
NW = NC * NS
CH = 128
NCHUNK = E // CH
NPAD = 10240
DEGW = 16

@functools.cache
def _vmesh():
    return plsc.VectorSubcoreMesh(core_axis_name="core", subcore_axis_name="subcore")


def _sc_params():
    cp = pltpu.CompilerParams()
    if "needs_layout_passes" in pltpu.CompilerParams.__dataclass_fields__:
        cp = dataclasses.replace(cp, needs_layout_passes=False)
    return cp


def _sc_degrees(edge_index, zeros_col):
    NFULL = (E // NW) // CH
    TAIL = E // NW - NFULL * CH

    @pl.kernel(
        out_type=jax.ShapeDtypeStruct((2, NW, NPAD), jnp.float32),
        mesh=_vmesh(),
        compiler_params=_sc_params(),
        scratch_types=[
            pltpu.VMEM((NPAD,), jnp.float32),
            pltpu.VMEM((NPAD,), jnp.float32),
            pltpu.VMEM((CH,), jnp.int32),
            pltpu.VMEM((CH,), jnp.int32),
            pltpu.VMEM((CH,), jnp.int32),
            pltpu.VMEM((CH,), jnp.int32),
            pltpu.VMEM((16,), jnp.int32),
            pltpu.VMEM((16,), jnp.int32),
            pltpu.SemaphoreType.DMA,
            pltpu.SemaphoreType.DMA,
        ],
    )
    def deg_kernel(edge_hbm, zero_hbm, out_hbm, hs, hd,
                   sidx0, didx0, sidx1, didx1, sidxt, didxt,
                   semi0, semi1):
        c = lax.axis_index("core")
        s = lax.axis_index("subcore")
        wid = c * NS + s
        pltpu.sync_copy(zero_hbm, hs)
        pltpu.sync_copy(zero_hbm, hd)
        ones16 = jnp.ones((16,), jnp.float32)
        base_w = wid * (E // NW)

        def accumulate(sbuf, dbuf, n16):
            for j in range(n16):
                plsc.addupdate_scatter(hs, [sbuf[pl.ds(j * 16, 16)]], ones16)
                plsc.addupdate_scatter(hd, [dbuf[pl.ds(j * 16, 16)]], ones16)

        def body(g, carry):
            b0 = base_w + (2 * g) * CH
            b1 = b0 + CH
            w0a = pltpu.async_copy(edge_hbm.at[pl.ds(b0, CH)], sidx0, semi0)
            w0b = pltpu.async_copy(edge_hbm.at[pl.ds(E + b0, CH)], didx0, semi0)
            w1a = pltpu.async_copy(edge_hbm.at[pl.ds(b1, CH)], sidx1, semi1)
            w1b = pltpu.async_copy(edge_hbm.at[pl.ds(E + b1, CH)], didx1, semi1)
            w0a.wait()
            w0b.wait()
            accumulate(sidx0, didx0, CH // 16)
            w1a.wait()
            w1b.wait()
            accumulate(sidx1, didx1, CH // 16)
            return carry

        lax.fori_loop(0, NFULL // 2, body, 0)

        bt = base_w + NFULL * CH
        pltpu.sync_copy(edge_hbm.at[pl.ds(bt, TAIL)], sidxt)
        pltpu.sync_copy(edge_hbm.at[pl.ds(E + bt, TAIL)], didxt)
        accumulate(sidxt, didxt, TAIL // 16)

        pltpu.sync_copy(hs, out_hbm.at[0, wid])
        pltpu.sync_copy(hd, out_hbm.at[1, wid])

    return deg_kernel(jnp.reshape(edge_index, (2 * E,)), zeros_col)


def _sc_aggregate(table, edge_index, zeros_rows):
    NFULL = (E // NW) // CH
    TAIL = E // NW - NFULL * CH

    @pl.kernel(
        out_type=jax.ShapeDtypeStruct((NC, NPAD, D), jnp.float32),
        mesh=_vmesh(),
        scratch_types=[
            pltpu.VMEM_SHARED((NPAD, D), jnp.float32),
            pltpu.VMEM((CH,), jnp.int32),
            pltpu.VMEM((CH,), jnp.int32),
            pltpu.VMEM((CH,), jnp.int32),
            pltpu.VMEM((CH,), jnp.int32),
            pltpu.VMEM((CH,), jnp.int32),
            pltpu.VMEM((CH,), jnp.int32),
            pltpu.VMEM((CH,), jnp.int32),
            pltpu.VMEM((CH,), jnp.int32),
            pltpu.VMEM((CH, D), jnp.float32),
            pltpu.VMEM((CH, D), jnp.float32),
            pltpu.VMEM((TAIL,), jnp.int32),
            pltpu.VMEM((TAIL,), jnp.int32),
            pltpu.VMEM((TAIL, D), jnp.float32),
            pltpu.SemaphoreType.DMA,
            pltpu.SemaphoreType.DMA,
            pltpu.SemaphoreType.DMA,
            pltpu.SemaphoreType.DMA,
            pltpu.SemaphoreType.DMA,
            pltpu.SemaphoreType.DMA,
            pltpu.SemaphoreType.DMA,
            pltpu.SemaphoreType.DMA,
        ],
    )
    def agg_kernel(table_hbm, edge_hbm, zero_hbm, out_hbm, acc_sh,
                   sidx0, didx0, sidx1, didx1, sidx2, didx2, sidx3, didx3,
                   rows0, rows1,
                   sidxt, didxt, rowst,
                   semi0, semi1, semi2, semi3,
                   semg0, semg1, semg2, semg3):
        c = lax.axis_index("core")
        s = lax.axis_index("subcore")
        wid = c * NS + s
        rows = NPAD // NS
        pltpu.sync_copy(zero_hbm, acc_sh.at[pl.ds(s * rows, rows)])
        plsc.subcore_barrier()

        base_w = wid * (E // NW)

        sbufs = (sidx0, sidx1, sidx2, sidx3)
        dbufs = (didx0, didx1, didx2, didx3)
        rbufs = (rows0, rows1)
        isems = (semi0, semi1, semi2, semi3)
        gsems = (semg0, semg1, semg2, semg3)

        def run(nch, g, carry):
            base = base_w + (4 * g) * CH
            iwaits = []
            for j in range(nch):
                b = base + j * CH
                iwaits.append(pltpu.async_copy(
                    edge_hbm.at[pl.ds(b, CH)], sbufs[j], isems[j]))
                iwaits.append(pltpu.async_copy(
                    edge_hbm.at[pl.ds(E + b, CH)], dbufs[j], isems[j]))
            gwaits = [None] * nch
            for j in range(nch):
                iwaits[2 * j].wait()
                iwaits[2 * j + 1].wait()
                if j >= 2:
                    gwaits[j - 2].wait()
                    pltpu.sync_copy(rbufs[(j - 2) % 2],
                                    acc_sh.at[dbufs[j - 2]], add=True)
                gwaits[j] = pltpu.async_copy(
                    table_hbm.at[sbufs[j]], rbufs[j % 2], gsems[j])
            for j in range(max(nch - 2, 0), nch):
                gwaits[j].wait()
                pltpu.sync_copy(rbufs[j % 2], acc_sh.at[dbufs[j]], add=True)
            return carry

        lax.fori_loop(0, NFULL // 4, functools.partial(run, 4), 0)
        run(NFULL - 4 * (NFULL // 4), NFULL // 4, 0)

        bt = base_w + NFULL * CH
        pltpu.sync_copy(edge_hbm.at[pl.ds(bt, TAIL)], sidxt)
        pltpu.sync_copy(edge_hbm.at[pl.ds(E + bt, TAIL)], didxt)
        pltpu.sync_copy(table_hbm.at[sidxt], rowst)
        pltpu.sync_copy(rowst, acc_sh.at[didxt], add=True)

        plsc.subcore_barrier()
        pltpu.sync_copy(acc_sh.at[pl.ds(s * rows, rows)],
                        out_hbm.at[c, pl.ds(s * rows, rows)])

    return agg_kernel(table, jnp.reshape(edge_index, (2 * E,)), zeros_rows)


_BR = 1000


def _tc_first_body(deg_ref, x_ref, w_ref, h_ref, ns_ref, nd_ref):
    degp = deg_ref[...]
    dsrc = jnp.sum(degp[0], axis=1, keepdims=True)
    ddst = jnp.sum(degp[1], axis=1, keepdims=True)
    ns = lax.rsqrt(jnp.maximum(dsrc, 1.0))
    nd = lax.rsqrt(jnp.maximum(ddst, 1.0))
    h = jnp.dot(x_ref[...], w_ref[...], preferred_element_type=jnp.float32)
    h_ref[...] = h * ns
    ns_ref[...] = ns
    nd_ref[...] = nd


def _tc_first(deg_p, x, W1):
    grid = (N // _BR,)
    return pl.pallas_call(
        _tc_first_body,
        grid=grid,
        in_specs=[
            pl.BlockSpec((2, _BR, NW), lambda i: (0, i, 0)),
            pl.BlockSpec((_BR, D), lambda i: (i, 0)),
            pl.BlockSpec((D, D), lambda i: (0, 0)),
        ],
        out_specs=[
            pl.BlockSpec((_BR, D), lambda i: (i, 0)),
            pl.BlockSpec((_BR, 1), lambda i: (i, 0)),
            pl.BlockSpec((_BR, 1), lambda i: (i, 0)),
        ],
        out_shape=[
            jax.ShapeDtypeStruct((N, D), jnp.float32),
            jax.ShapeDtypeStruct((N, 1), jnp.float32),
            jax.ShapeDtypeStruct((N, 1), jnp.float32),
        ],
    )(deg_p, x, W1)


def _tc_mid_body(p_ref, nd_ref, ns_ref, w_ref, b_ref, h_ref):
    agg = p_ref[0] + p_ref[1]
    x1 = jnp.maximum(agg * nd_ref[...] + b_ref[...], 0.0)
    h = jnp.dot(x1, w_ref[...], preferred_element_type=jnp.float32)
    h_ref[...] = h * ns_ref[...]


def _tc_mid(p1, nd, ns, W2, b1_row):
    grid = (N // _BR,)
    return pl.pallas_call(
        _tc_mid_body,
        grid=grid,
        in_specs=[
            pl.BlockSpec((NC, _BR, D), lambda i: (0, i, 0)),
            pl.BlockSpec((_BR, 1), lambda i: (i, 0)),
            pl.BlockSpec((_BR, 1), lambda i: (i, 0)),
            pl.BlockSpec((D, D), lambda i: (0, 0)),
            pl.BlockSpec((1, D), lambda i: (0, 0)),
        ],
        out_specs=pl.BlockSpec((_BR, D), lambda i: (i, 0)),
        out_shape=jax.ShapeDtypeStruct((N, D), jnp.float32),
    )(p1, nd, ns, W2, b1_row)


def _tc_final_body(p_ref, nd_ref, b_ref, o_ref):
    o_ref[...] = (p_ref[0] + p_ref[1]) * nd_ref[...] + b_ref[...]


def _tc_final(p2, nd, b2_row):
    grid = (N // _BR,)
    return pl.pallas_call(
        _tc_final_body,
        grid=grid,
        in_specs=[
            pl.BlockSpec((NC, _BR, D), lambda i: (0, i, 0)),
            pl.BlockSpec((_BR, 1), lambda i: (i, 0)),
            pl.BlockSpec((1, D), lambda i: (0, 0)),
        ],
        out_specs=pl.BlockSpec((_BR, D), lambda i: (i, 0)),
        out_shape=jax.ShapeDtypeStruct((N, D), jnp.float32),
    )(p2, nd, b2_row)


def kernel(edge_index, node_embeddings, W1, b1, W2, b2):
    zeros_col = jnp.zeros((NPAD,), jnp.float32)
    zeros_rows = jnp.zeros((NPAD // NS, D), jnp.float32)

    deg_p = jnp.transpose(_sc_degrees(edge_index, zeros_col), (0, 2, 1))
    h1, ns, nd = _tc_first(deg_p, node_embeddings, W1)
    p1 = _sc_aggregate(h1, edge_index, zeros_rows)
    h2 = _tc_mid(p1, nd, ns, W2, jnp.reshape(b1, (1, D)))
    p2 = _sc_aggregate(h2, edge_index, zeros_rows)
    return _tc_final(p2, nd, jnp.reshape(b2, (1, D)))


# speedup vs baseline: 16.8006x; 1.0418x over previous
import dataclasses
import functools

import jax
import jax.numpy as jnp
from jax import lax
from jax.experimental import pallas as pl
from jax.experimental.pallas import tpu as pltpu
from jax.experimental.pallas import tpu_sc as plsc

N = 10000
D = 128
E = 320000

NC = 2
NS = 16
NW = NC * NS
CH = 128
NCHUNK = E // CH
NPAD = 10240
DEGW = 16

@functools.cache
def _vmesh():
    return plsc.VectorSubcoreMesh(core_axis_name="core", subcore_axis_name="subcore")


def _sc_params():
    cp = pltpu.CompilerParams()
    if "needs_layout_passes" in pltpu.CompilerParams.__dataclass_fields__:
        cp = dataclasses.replace(cp, needs_layout_passes=False)
    return cp


def _sc_degrees(edge_index, zeros_col):
    NFULL = (E // NW) // CH
    TAIL = E // NW - NFULL * CH

    @pl.kernel(
        out_type=jax.ShapeDtypeStruct((2, NW, NPAD), jnp.float32),
        mesh=_vmesh(),
        compiler_params=_sc_params(),
        scratch_types=[
            pltpu.VMEM((NPAD,), jnp.float32),
            pltpu.VMEM((NPAD,), jnp.float32),
            pltpu.VMEM((CH,), jnp.int32),
            pltpu.VMEM((CH,), jnp.int32),
            pltpu.VMEM((CH,), jnp.int32),
            pltpu.VMEM((CH,), jnp.int32),
            pltpu.VMEM((16,), jnp.int32),
            pltpu.VMEM((16,), jnp.int32),
            pltpu.SemaphoreType.DMA,
            pltpu.SemaphoreType.DMA,
        ],
    )
    def deg_kernel(edge_hbm, zero_hbm, out_hbm, hs, hd,
                   sidx0, didx0, sidx1, didx1, sidxt, didxt,
                   semi0, semi1):
        c = lax.axis_index("core")
        s = lax.axis_index("subcore")
        wid = c * NS + s
        pltpu.sync_copy(zero_hbm, hs)
        pltpu.sync_copy(zero_hbm, hd)
        ones16 = jnp.ones((16,), jnp.float32)
        base_w = wid * (E // NW)

        def accumulate(sbuf, dbuf, n16):
            for j in range(n16):
                plsc.addupdate_scatter(hs, [sbuf[pl.ds(j * 16, 16)]], ones16)
                plsc.addupdate_scatter(hd, [dbuf[pl.ds(j * 16, 16)]], ones16)

        def body(g, carry):
            b0 = base_w + (2 * g) * CH
            b1 = b0 + CH
            w0a = pltpu.async_copy(edge_hbm.at[pl.ds(b0, CH)], sidx0, semi0)
            w0b = pltpu.async_copy(edge_hbm.at[pl.ds(E + b0, CH)], didx0, semi0)
            w1a = pltpu.async_copy(edge_hbm.at[pl.ds(b1, CH)], sidx1, semi1)
            w1b = pltpu.async_copy(edge_hbm.at[pl.ds(E + b1, CH)], didx1, semi1)
            w0a.wait()
            w0b.wait()
            accumulate(sidx0, didx0, CH // 16)
            w1a.wait()
            w1b.wait()
            accumulate(sidx1, didx1, CH // 16)
            return carry

        lax.fori_loop(0, NFULL // 2, body, 0)

        bt = base_w + NFULL * CH
        pltpu.sync_copy(edge_hbm.at[pl.ds(bt, TAIL)], sidxt)
        pltpu.sync_copy(edge_hbm.at[pl.ds(E + bt, TAIL)], didxt)
        accumulate(sidxt, didxt, TAIL // 16)

        pltpu.sync_copy(hs, out_hbm.at[0, wid])
        pltpu.sync_copy(hd, out_hbm.at[1, wid])

    return deg_kernel(jnp.reshape(edge_index, (2 * E,)), zeros_col)


def _sc_aggregate(table, edge_index, zeros_rows):
    ACH = 64
    GRP = 8
    NRB = 4
    PER_W = E // NW
    NFULL = PER_W // ACH
    NGRP = NFULL // GRP
    REM = NFULL - NGRP * GRP
    TAIL = PER_W - NFULL * ACH

    @pl.kernel(
        out_type=jax.ShapeDtypeStruct((NC, NPAD, D), jnp.float32),
        mesh=_vmesh(),
        scratch_types=(
            [pltpu.VMEM_SHARED((NPAD, D), jnp.float32)]
            + [pltpu.VMEM((ACH,), jnp.int32)] * (2 * GRP)
            + [pltpu.VMEM((ACH, D), jnp.float32)] * NRB
            + [pltpu.VMEM((TAIL,), jnp.int32)] * 2
            + [pltpu.VMEM((TAIL, D), jnp.float32)]
            + [pltpu.SemaphoreType.DMA] * (GRP + NRB)
        ),
    )
    def agg_kernel(table_hbm, edge_hbm, zero_hbm, out_hbm, acc_sh, *bufs):
        sbufs = bufs[0:GRP]
        dbufs = bufs[GRP:2 * GRP]
        rbufs = bufs[2 * GRP:2 * GRP + NRB]
        sidxt = bufs[2 * GRP + NRB]
        didxt = bufs[2 * GRP + NRB + 1]
        rowst = bufs[2 * GRP + NRB + 2]
        isems = bufs[2 * GRP + NRB + 3:2 * GRP + NRB + 3 + GRP]
        gsems = bufs[2 * GRP + NRB + 3 + GRP:]

        c = lax.axis_index("core")
        s = lax.axis_index("subcore")
        wid = c * NS + s
        rows = NPAD // NS
        pltpu.sync_copy(zero_hbm, acc_sh.at[pl.ds(s * rows, rows)])
        plsc.subcore_barrier()

        base_w = wid * PER_W

        def run(nch, g, carry):
            base = base_w + (GRP * g) * ACH
            iwaits = []
            for j in range(nch):
                b = base + j * ACH
                iwaits.append(pltpu.async_copy(
                    edge_hbm.at[pl.ds(b, ACH)], sbufs[j], isems[j]))
                iwaits.append(pltpu.async_copy(
                    edge_hbm.at[pl.ds(E + b, ACH)], dbufs[j], isems[j]))
            gwaits = [None] * nch
            for j in range(nch):
                iwaits[2 * j].wait()
                iwaits[2 * j + 1].wait()
                if j >= NRB:
                    gwaits[j - NRB].wait()
                    pltpu.sync_copy(rbufs[(j - NRB) % NRB],
                                    acc_sh.at[dbufs[j - NRB]], add=True)
                gwaits[j] = pltpu.async_copy(
                    table_hbm.at[sbufs[j]], rbufs[j % NRB], gsems[j % NRB])
            for j in range(max(nch - NRB, 0), nch):
                gwaits[j].wait()
                pltpu.sync_copy(rbufs[j % NRB], acc_sh.at[dbufs[j]], add=True)
            return carry

        lax.fori_loop(0, NGRP, functools.partial(run, GRP), 0)
        run(REM, NGRP, 0)

        bt = base_w + NFULL * ACH
        pltpu.sync_copy(edge_hbm.at[pl.ds(bt, TAIL)], sidxt)
        pltpu.sync_copy(edge_hbm.at[pl.ds(E + bt, TAIL)], didxt)
        pltpu.sync_copy(table_hbm.at[sidxt], rowst)
        pltpu.sync_copy(rowst, acc_sh.at[didxt], add=True)

        plsc.subcore_barrier()
        pltpu.sync_copy(acc_sh.at[pl.ds(s * rows, rows)],
                        out_hbm.at[c, pl.ds(s * rows, rows)])

    return agg_kernel(table, jnp.reshape(edge_index, (2 * E,)), zeros_rows)


_BR = 1000


def _tc_first_body(deg_ref, x_ref, w_ref, h_ref, ns_ref, nd_ref):
    degp = deg_ref[...]
    dsrc = jnp.sum(degp[0], axis=1, keepdims=True)
    ddst = jnp.sum(degp[1], axis=1, keepdims=True)
    ns = lax.rsqrt(jnp.maximum(dsrc, 1.0))
    nd = lax.rsqrt(jnp.maximum(ddst, 1.0))
    h = jnp.dot(x_ref[...], w_ref[...], preferred_element_type=jnp.float32)
    h_ref[...] = h * ns
    ns_ref[...] = ns
    nd_ref[...] = nd


def _tc_first(deg_p, x, W1):
    grid = (N // _BR,)
    return pl.pallas_call(
        _tc_first_body,
        grid=grid,
        in_specs=[
            pl.BlockSpec((2, _BR, NW), lambda i: (0, i, 0)),
            pl.BlockSpec((_BR, D), lambda i: (i, 0)),
            pl.BlockSpec((D, D), lambda i: (0, 0)),
        ],
        out_specs=[
            pl.BlockSpec((_BR, D), lambda i: (i, 0)),
            pl.BlockSpec((_BR, 1), lambda i: (i, 0)),
            pl.BlockSpec((_BR, 1), lambda i: (i, 0)),
        ],
        out_shape=[
            jax.ShapeDtypeStruct((N, D), jnp.float32),
            jax.ShapeDtypeStruct((N, 1), jnp.float32),
            jax.ShapeDtypeStruct((N, 1), jnp.float32),
        ],
    )(deg_p, x, W1)


def _tc_mid_body(p_ref, nd_ref, ns_ref, w_ref, b_ref, h_ref):
    agg = p_ref[0] + p_ref[1]
    x1 = jnp.maximum(agg * nd_ref[...] + b_ref[...], 0.0)
    h = jnp.dot(x1, w_ref[...], preferred_element_type=jnp.float32)
    h_ref[...] = h * ns_ref[...]


def _tc_mid(p1, nd, ns, W2, b1_row):
    grid = (N // _BR,)
    return pl.pallas_call(
        _tc_mid_body,
        grid=grid,
        in_specs=[
            pl.BlockSpec((NC, _BR, D), lambda i: (0, i, 0)),
            pl.BlockSpec((_BR, 1), lambda i: (i, 0)),
            pl.BlockSpec((_BR, 1), lambda i: (i, 0)),
            pl.BlockSpec((D, D), lambda i: (0, 0)),
            pl.BlockSpec((1, D), lambda i: (0, 0)),
        ],
        out_specs=pl.BlockSpec((_BR, D), lambda i: (i, 0)),
        out_shape=jax.ShapeDtypeStruct((N, D), jnp.float32),
    )(p1, nd, ns, W2, b1_row)


def _tc_final_body(p_ref, nd_ref, b_ref, o_ref):
    o_ref[...] = (p_ref[0] + p_ref[1]) * nd_ref[...] + b_ref[...]


def _tc_final(p2, nd, b2_row):
    grid = (N // _BR,)
    return pl.pallas_call(
        _tc_final_body,
        grid=grid,
        in_specs=[
            pl.BlockSpec((NC, _BR, D), lambda i: (0, i, 0)),
            pl.BlockSpec((_BR, 1), lambda i: (i, 0)),
            pl.BlockSpec((1, D), lambda i: (0, 0)),
        ],
        out_specs=pl.BlockSpec((_BR, D), lambda i: (i, 0)),
        out_shape=jax.ShapeDtypeStruct((N, D), jnp.float32),
    )(p2, nd, b2_row)


def kernel(edge_index, node_embeddings, W1, b1, W2, b2):
    zeros_col = jnp.zeros((NPAD,), jnp.float32)
    zeros_rows = jnp.zeros((NPAD // NS, D), jnp.float32)

    deg_p = jnp.transpose(_sc_degrees(edge_index, zeros_col), (0, 2, 1))
    h1, ns, nd = _tc_first(deg_p, node_embeddings, W1)
    p1 = _sc_aggregate(h1, edge_index, zeros_rows)
    h2 = _tc_mid(p1, nd, ns, W2, jnp.reshape(b1, (1, D)))
    p2 = _sc_aggregate(h2, edge_index, zeros_rows)
    return _tc_final(p2, nd, jnp.reshape(b2, (1, D)))
